# baseline (speedup 1.0000x reference)
import functools

import jax
import jax.numpy as jnp
from jax import lax
from jax.experimental import pallas as pl
from jax.experimental.pallas import tpu as pltpu
from jax.experimental.pallas import tpu_sc as plsc

_DPAD = 8
_PACK = 128 // _DPAD


_ROWS = 8192


def _tw_body(scale, nblocks, tablet_hbm, w_ref, b_ref, out_ref, tbuf, sem):
    i = pl.program_id(0)
    slot = lax.rem(i, 2)

    def issue(j, s):
        pltpu.make_async_copy(
            tablet_hbm.at[:, pl.ds(j * _ROWS, _ROWS)], tbuf.at[s], sem.at[s]
        ).start()

    @pl.when(i == 0)
    def _():
        issue(0, 0)

    @pl.when(i + 1 < nblocks)
    def _():
        issue(i + 1, 1 - slot)

    pltpu.make_async_copy(
        tablet_hbm.at[:, pl.ds(0, _ROWS)], tbuf.at[slot], sem.at[slot]
    ).wait()
    w = w_ref[...] * scale
    tw = (
        lax.dot_general(
            tbuf[slot],
            w,
            dimension_numbers=(((0,), (0,)), ((), ())),
            preferred_element_type=jnp.float32,
        )
        + b_ref[...] * scale
    )
    c = tw.shape[1]
    out_ref[...] = jnp.concatenate(
        [
            tw.astype(jnp.bfloat16),
            jnp.zeros((_ROWS, 128 - c), jnp.bfloat16),
        ],
        axis=1,
    )


def _make_tw_main(vocab, d, c, seq_len):
    nblocks = (vocab // _ROWS)
    return pl.pallas_call(
        functools.partial(_tw_body, 1.0 / float(seq_len), nblocks),
        grid=(nblocks,),
        in_specs=[
            pl.BlockSpec(memory_space=pl.ANY),
            pl.BlockSpec((d, c), lambda i: (0, 0)),
            pl.BlockSpec((1, c), lambda i: (0, 0)),
        ],
        out_specs=pl.BlockSpec((_ROWS, 128), lambda i: (i, 0)),
        out_shape=jax.ShapeDtypeStruct((vocab, 128), jnp.bfloat16),
        scratch_shapes=[
            pltpu.VMEM((2, d, _ROWS), jnp.float32),
            pltpu.SemaphoreType.DMA((2,)),
        ],
        compiler_params=pltpu.CompilerParams(
            fuse_transposed_lhs_in_matmul=True
        ),
    )


def _tail_body(scale, alias_ref, ttail_ref, w_ref, b_ref, out_ref):
    w = w_ref[...] * scale
    tw = (
        jnp.dot(ttail_ref[...], w, preferred_element_type=jnp.float32)
        + b_ref[...] * scale
    )
    rows, c = tw.shape
    out_ref[...] = jnp.concatenate(
        [
            tw.astype(jnp.bfloat16),
            jnp.zeros((rows, 128 - c), jnp.bfloat16),
        ],
        axis=1,
    )


def _make_tw_tail(vocab, d, c, seq_len, tail):
    base_blk = (vocab - tail) // 64
    return pl.pallas_call(
        functools.partial(_tail_body, 1.0 / float(seq_len)),
        grid=(tail // 64,),
        in_specs=[
            pl.BlockSpec(memory_space=pl.ANY),
            pl.BlockSpec((64, d), lambda i: (i, 0)),
            pl.BlockSpec((d, c), lambda i: (0, 0)),
            pl.BlockSpec((1, c), lambda i: (0, 0)),
        ],
        out_specs=pl.BlockSpec((64, 128), lambda i: (base_blk + i, 0)),
        out_shape=jax.ShapeDtypeStruct((vocab, 128), jnp.bfloat16),
        input_output_aliases={0: 0},
    )


def _lane_gather(v, idx):
    dn = lax.GatherDimensionNumbers(
        offset_dims=(), collapsed_slice_dims=(0,), start_index_map=(0,)
    )
    return lax.gather(
        v,
        idx[:, None],
        dn,
        slice_sizes=(1,),
        mode=lax.GatherScatterMode.PROMISE_IN_BOUNDS,
    )


def _make_sc(batch, seq_len, c, vocab):
    info = plsc.get_sparse_core_info()
    nc, ns = info.num_cores, info.num_subcores
    nw = nc * ns
    rpw = batch // nw
    cr = 16
    n_chunks = rpw // cr
    tpc = cr * seq_len
    steps = seq_len // 2

    mesh = plsc.VectorSubcoreMesh(core_axis_name="c", subcore_axis_name="s")

    @functools.partial(
        pl.kernel,
        mesh=mesh,
        out_type=jax.ShapeDtypeStruct((batch * c,), jnp.float32),
        scratch_types=[
            pltpu.VMEM((cr, seq_len), jnp.int32),
            pltpu.VMEM((cr, seq_len), jnp.int32),
            pltpu.VMEM((tpc, 32), jnp.bfloat16),
            pltpu.VMEM((tpc, 32), jnp.bfloat16),
            pltpu.VMEM((rpw * c,), jnp.float32),
            pltpu.SemaphoreType.DMA,
            pltpu.SemaphoreType.DMA,
            pltpu.SemaphoreType.DMA,
            pltpu.SemaphoreType.DMA,
        ],
        compiler_params=pltpu.CompilerParams(
            needs_layout_passes=False, use_tc_tiling_on_sc=False
        ),
    )
    def sc_kernel(
        x_hbm, tw_hbm, out_hbm,
        ib0, ib1, db0, db1, outbuf,
        sem_i0, sem_i1, sem_g0, sem_g1,
    ):
        wid = lax.axis_index("s") * nc + lax.axis_index("c")
        row0 = wid * rpw
        iota = lax.iota(jnp.int32, 16)
        apat = lax.shift_left(lax.bitwise_and(iota, 1), 1)
        amask = iota < 2
        himask = jnp.full((16,), jnp.int32(-65536))

        def issue_idx(g, ib, sem):
            row = row0 + g * cr
            pltpu.async_copy(x_hbm.at[pl.ds(row, cr), :], ib, sem)

        def wait_idx(ib, sem):
            pltpu.make_async_copy(x_hbm.at[pl.ds(0, cr), :], ib, sem).wait()

        def issue_gather(ib, db, sem):
            def body(r, _):
                pltpu.async_copy(
                    tw_hbm.at[ib.at[r]],
                    db.at[pl.ds(r * seq_len, seq_len), :],
                    sem,
                )
                return 0

            lax.fori_loop(0, cr, body, 0)

        def wait_gather(ib, db, sem):
            def body(r, _):
                pltpu.make_async_copy(
                    tw_hbm.at[ib.at[r]],
                    db.at[pl.ds(r * seq_len, seq_len), :],
                    sem,
                ).wait()
                return 0

            lax.fori_loop(0, cr, body, 0)

        def accum(g, db):
            def row_body(r, _):
                base = r * seq_len
                acc_a = jnp.zeros((16,), jnp.float32)
                acc_b = jnp.zeros((16,), jnp.float32)
                for s in range(steps):
                    v = db[base + 2 * s] + db[base + 2 * s + 1]
                    vi = plsc.bitcast(v, jnp.int32)
                    acc_a = acc_a + plsc.bitcast(
                        lax.shift_left(vi, 16), jnp.float32
                    )
                    acc_b = acc_b + plsc.bitcast(
                        lax.bitwise_and(vi, himask), jnp.float32
                    )
                obase = (g * cr + r) * c
                plsc.store_scatter(outbuf, [obase + apat], acc_a, mask=amask)
                plsc.store_scatter(
                    outbuf, [obase + 1 + apat], acc_b, mask=amask
                )
                return 0

            lax.fori_loop(0, cr, row_body, 0)

        issue_idx(0, ib0, sem_i0)
        issue_idx(1, ib1, sem_i1)
        wait_idx(ib0, sem_i0)
        issue_gather(ib0, db0, sem_g0)

        def body2(k, _):
            g0 = 2 * k
            wait_gather(ib0, db0, sem_g0)

            @pl.when(g0 + 2 < n_chunks)
            def _():
                issue_idx(g0 + 2, ib0, sem_i0)

            wait_idx(ib1, sem_i1)
            issue_gather(ib1, db1, sem_g1)
            accum(g0, db0)

            wait_gather(ib1, db1, sem_g1)

            @pl.when(g0 + 3 < n_chunks)
            def _():
                issue_idx(g0 + 3, ib1, sem_i1)

            @pl.when(g0 + 2 < n_chunks)
            def _():
                wait_idx(ib0, sem_i0)
                issue_gather(ib0, db0, sem_g0)

            accum(g0 + 1, db1)
            return 0

        lax.fori_loop(0, n_chunks // 2, body2, 0)
        pltpu.sync_copy(outbuf, out_hbm.at[pl.ds(row0 * c, rpw * c)])

    return sc_kernel


def kernel(x, table, W, b):
    batch, seq_len = x.shape
    vocab, d = table.shape
    c = W.shape[1]
    b1 = b.reshape(1, c)
    tail = vocab - (vocab // _ROWS) * _ROWS
    tw128 = _make_tw_main(vocab, d, c, seq_len)(table.T, W, b1)
    if tail:
        ttail = lax.slice(table, (vocab - tail, 0), (vocab, d))
        tw128 = _make_tw_tail(vocab, d, c, seq_len, tail)(
            tw128, ttail, W, b1
        )
    tw = tw128.reshape(vocab * 4, 32)
    x4 = x * 4
    out_flat = _make_sc(batch, seq_len, c, vocab)(x4, tw)
    return out_flat.reshape(batch, c)


# speedup vs baseline: 2.7240x; 2.7240x over previous
import functools

import jax
import jax.numpy as jnp
from jax import lax
from jax.experimental import pallas as pl
from jax.experimental.pallas import tpu as pltpu
from jax.experimental.pallas import tpu_sc as plsc

_DPAD = 8
_ROWS = 8192


def _tw_body(scale, nblocks, tablet_hbm, w_ref, b_ref, out_ref, tbuf, sem):
    i = pl.program_id(0)
    slot = lax.rem(i, 2)

    def issue(j, s):
        pltpu.make_async_copy(
            tablet_hbm.at[:, pl.ds(j * _ROWS, _ROWS)], tbuf.at[s], sem.at[s]
        ).start()

    @pl.when(i == 0)
    def _():
        issue(0, 0)

    @pl.when(i + 1 < nblocks)
    def _():
        issue(i + 1, 1 - slot)

    pltpu.make_async_copy(
        tablet_hbm.at[:, pl.ds(0, _ROWS)], tbuf.at[slot], sem.at[slot]
    ).wait()
    w = w_ref[...] * scale
    tw = (
        lax.dot_general(
            tbuf[slot],
            w,
            dimension_numbers=(((0,), (0,)), ((), ())),
            preferred_element_type=jnp.float32,
        )
        + b_ref[...] * scale
    )
    c = tw.shape[1]
    out_ref[...] = jnp.concatenate(
        [tw, jnp.zeros((_ROWS, 128 - c), jnp.float32)], axis=1
    )


def _make_tw_main(vocab, d, c, seq_len):
    nblocks = vocab // _ROWS
    return pl.pallas_call(
        functools.partial(_tw_body, 1.0 / float(seq_len), nblocks),
        grid=(nblocks,),
        in_specs=[
            pl.BlockSpec(memory_space=pl.ANY),
            pl.BlockSpec((d, c), lambda i: (0, 0)),
            pl.BlockSpec((1, c), lambda i: (0, 0)),
        ],
        out_specs=pl.BlockSpec((_ROWS, 128), lambda i: (i, 0)),
        out_shape=jax.ShapeDtypeStruct((vocab, 128), jnp.float32),
        scratch_shapes=[
            pltpu.VMEM((2, d, _ROWS), jnp.float32),
            pltpu.SemaphoreType.DMA((2,)),
        ],
        compiler_params=pltpu.CompilerParams(
            fuse_transposed_lhs_in_matmul=True
        ),
    )


def _tail_body(scale, alias_ref, ttail_ref, w_ref, b_ref, out_ref):
    w = w_ref[...] * scale
    tw = (
        jnp.dot(ttail_ref[...], w, preferred_element_type=jnp.float32)
        + b_ref[...] * scale
    )
    rows, c = tw.shape
    out_ref[...] = jnp.concatenate(
        [tw, jnp.zeros((rows, 128 - c), jnp.float32)], axis=1
    )


def _make_tw_tail(vocab, d, c, seq_len, tail):
    base_blk = (vocab - tail) // 64
    return pl.pallas_call(
        functools.partial(_tail_body, 1.0 / float(seq_len)),
        grid=(tail // 64,),
        in_specs=[
            pl.BlockSpec(memory_space=pl.ANY),
            pl.BlockSpec((64, d), lambda i: (i, 0)),
            pl.BlockSpec((d, c), lambda i: (0, 0)),
            pl.BlockSpec((1, c), lambda i: (0, 0)),
        ],
        out_specs=pl.BlockSpec((64, 128), lambda i: (base_blk + i, 0)),
        out_shape=jax.ShapeDtypeStruct((vocab, 128), jnp.float32),
        input_output_aliases={0: 0},
    )


def _lane_gather(v, idx):
    dn = lax.GatherDimensionNumbers(
        offset_dims=(), collapsed_slice_dims=(0,), start_index_map=(0,)
    )
    return lax.gather(
        v,
        idx[:, None],
        dn,
        slice_sizes=(1,),
        mode=lax.GatherScatterMode.PROMISE_IN_BOUNDS,
    )


def _make_sc(batch, seq_len, c, vocab):
    info = plsc.get_sparse_core_info()
    nc, ns = info.num_cores, info.num_subcores
    nw = nc * ns
    rpw = batch // nw
    cr = 16
    n_chunks = rpw // cr
    tpc = cr * seq_len
    steps = seq_len // 2

    mesh = plsc.VectorSubcoreMesh(core_axis_name="c", subcore_axis_name="s")

    @functools.partial(
        pl.kernel,
        mesh=mesh,
        out_type=jax.ShapeDtypeStruct((batch * c,), jnp.float32),
        scratch_types=[
            pltpu.VMEM((seq_len, cr), jnp.int32),
            pltpu.VMEM((seq_len, cr), jnp.int32),
            pltpu.VMEM((tpc,), jnp.int32),
            pltpu.VMEM((tpc,), jnp.int32),
            pltpu.VMEM((tpc, _DPAD), jnp.float32),
            pltpu.VMEM((tpc, _DPAD), jnp.float32),
            pltpu.VMEM((rpw * c,), jnp.float32),
            pltpu.SemaphoreType.DMA,
            pltpu.SemaphoreType.DMA,
            pltpu.SemaphoreType.DMA,
            pltpu.SemaphoreType.DMA,
        ],
        compiler_params=pltpu.CompilerParams(
            needs_layout_passes=False, use_tc_tiling_on_sc=False
        ),
    )
    def sc_kernel(
        xt_hbm, tw_hbm, out_hbm,
        ibt0, ibt1, ib0, ib1, db0, db1, outbuf,
        sem_t0, sem_t1, sem_g0, sem_g1,
    ):
        wid = lax.axis_index("s") * nc + lax.axis_index("c")
        row0 = wid * rpw
        iota = lax.iota(jnp.int32, 16)
        riota = lax.shift_right_logical(iota, 3)
        cpat = lax.bitwise_and(iota, 7)
        opat = lax.bitwise_and(iota, 3)
        fold8 = lax.bitwise_and(iota + 8, 15)
        omask = iota < 4
        riota200 = iota * seq_len

        def issue_ibt(g, ibt, sem):
            col = row0 + g * cr
            pltpu.async_copy(xt_hbm.at[:, pl.ds(col, cr)], ibt, sem)

        def wait_ibt(ibt, sem):
            pltpu.make_async_copy(
                xt_hbm.at[:, pl.ds(0, cr)], ibt, sem
            ).wait()

        def transpose(ibt, ib):
            def body(l, _):
                v = lax.shift_left(ibt[l], 4)
                plsc.store_scatter(ib, [riota200 + l], v)
                return 0

            lax.fori_loop(0, seq_len, body, 0)

        def issue_gathers(ib, db, sem):
            def body(r, _):
                pltpu.async_copy(
                    tw_hbm.at[ib.at[pl.ds(r * seq_len, seq_len)]],
                    db.at[pl.ds(r * seq_len, seq_len), :],
                    sem,
                )
                return 0

            lax.fori_loop(0, cr, body, 0)

        def wait_gathers(ib, db, sem):
            def body(r, _):
                pltpu.make_async_copy(
                    tw_hbm.at[ib.at[pl.ds(r * seq_len, seq_len)]],
                    db.at[pl.ds(r * seq_len, seq_len), :],
                    sem,
                ).wait()
                return 0

            lax.fori_loop(0, cr, body, 0)

        def accum(g, db):
            def row_body(r, _):
                base = r * seq_len
                acc = jnp.zeros((16,), jnp.float32)
                for s in range(steps):
                    ridx = riota + (base + 2 * s)
                    acc = acc + plsc.load_gather(db, [ridx, cpat])
                a1 = acc + _lane_gather(acc, fold8)
                oidx = (g * cr + r) * c + opat
                plsc.store_scatter(outbuf, [oidx], a1, mask=omask)
                return 0

            lax.fori_loop(0, cr, row_body, 0)

        issue_ibt(0, ibt0, sem_t0)
        wait_ibt(ibt0, sem_t0)
        transpose(ibt0, ib0)
        issue_gathers(ib0, db0, sem_g0)
        issue_ibt(1, ibt1, sem_t1)

        def body2(k, _):
            g0 = 2 * k
            wait_gathers(ib0, db0, sem_g0)
            wait_ibt(ibt1, sem_t1)
            transpose(ibt1, ib1)
            issue_gathers(ib1, db1, sem_g1)

            @pl.when(g0 + 2 < n_chunks)
            def _():
                issue_ibt(g0 + 2, ibt0, sem_t0)

            accum(g0, db0)

            wait_gathers(ib1, db1, sem_g1)

            @pl.when(g0 + 2 < n_chunks)
            def _():
                wait_ibt(ibt0, sem_t0)
                transpose(ibt0, ib0)
                issue_gathers(ib0, db0, sem_g0)

            @pl.when(g0 + 3 < n_chunks)
            def _():
                issue_ibt(g0 + 3, ibt1, sem_t1)

            accum(g0 + 1, db1)
            return 0

        lax.fori_loop(0, n_chunks // 2, body2, 0)
        pltpu.sync_copy(outbuf, out_hbm.at[pl.ds(row0 * c, rpw * c)])

    return sc_kernel


def kernel(x, table, W, b):
    batch, seq_len = x.shape
    vocab, d = table.shape
    c = W.shape[1]
    b1 = b.reshape(1, c)
    tail = vocab - (vocab // _ROWS) * _ROWS
    tw128 = _make_tw_main(vocab, d, c, seq_len)(table.T, W, b1)
    if tail:
        ttail = lax.slice(table, (vocab - tail, 0), (vocab, d))
        tw128 = _make_tw_tail(vocab, d, c, seq_len, tail)(
            tw128, ttail, W, b1
        )
    tw = tw128.reshape(vocab * (128 // _DPAD), _DPAD)
    out_flat = _make_sc(batch, seq_len, c, vocab)(x.T, tw)
    return out_flat.reshape(batch, c)


# speedup vs baseline: 2.9576x; 1.0858x over previous
import functools

import jax
import jax.numpy as jnp
from jax import lax
from jax.experimental import pallas as pl
from jax.experimental.pallas import tpu as pltpu
from jax.experimental.pallas import tpu_sc as plsc

_DPAD = 8
_ROWS = 16384


def _tw_body(scale, nblocks, tablet_hbm, w_ref, b_ref, out_ref, tbuf, sem):
    i = pl.program_id(0)
    slot = lax.rem(i, 2)

    def issue(j, s):
        pltpu.make_async_copy(
            tablet_hbm.at[:, pl.ds(j * _ROWS, _ROWS)], tbuf.at[s], sem.at[s]
        ).start()

    @pl.when(i == 0)
    def _():
        issue(0, 0)

    @pl.when(i + 1 < nblocks)
    def _():
        issue(i + 1, 1 - slot)

    pltpu.make_async_copy(
        tablet_hbm.at[:, pl.ds(0, _ROWS)], tbuf.at[slot], sem.at[slot]
    ).wait()
    w = w_ref[...] * scale
    tw = (
        lax.dot_general(
            tbuf[slot],
            w,
            dimension_numbers=(((0,), (0,)), ((), ())),
            preferred_element_type=jnp.float32,
        )
        + b_ref[...] * scale
    )
    c = tw.shape[1]
    out_ref[...] = jnp.concatenate(
        [tw, jnp.zeros((_ROWS, 128 - c), jnp.float32)], axis=1
    )


def _make_tw_main(vocab, d, c, seq_len):
    nblocks = vocab // _ROWS
    return pl.pallas_call(
        functools.partial(_tw_body, 1.0 / float(seq_len), nblocks),
        grid=(nblocks,),
        in_specs=[
            pl.BlockSpec(memory_space=pl.ANY),
            pl.BlockSpec((d, c), lambda i: (0, 0)),
            pl.BlockSpec((1, c), lambda i: (0, 0)),
        ],
        out_specs=pl.BlockSpec((_ROWS, 128), lambda i: (i, 0)),
        out_shape=jax.ShapeDtypeStruct((vocab, 128), jnp.float32),
        scratch_shapes=[
            pltpu.VMEM((2, d, _ROWS), jnp.float32),
            pltpu.SemaphoreType.DMA((2,)),
        ],
        compiler_params=pltpu.CompilerParams(
            fuse_transposed_lhs_in_matmul=True
        ),
    )


def _tail_body(scale, alias_ref, ttail_ref, w_ref, b_ref, out_ref):
    w = w_ref[...] * scale
    tw = (
        jnp.dot(ttail_ref[...], w, preferred_element_type=jnp.float32)
        + b_ref[...] * scale
    )
    rows, c = tw.shape
    out_ref[...] = jnp.concatenate(
        [tw, jnp.zeros((rows, 128 - c), jnp.float32)], axis=1
    )


def _make_tw_tail(vocab, d, c, seq_len, tail):
    base_blk = (vocab - tail) // 64
    return pl.pallas_call(
        functools.partial(_tail_body, 1.0 / float(seq_len)),
        grid=(tail // 64,),
        in_specs=[
            pl.BlockSpec(memory_space=pl.ANY),
            pl.BlockSpec((64, d), lambda i: (i, 0)),
            pl.BlockSpec((d, c), lambda i: (0, 0)),
            pl.BlockSpec((1, c), lambda i: (0, 0)),
        ],
        out_specs=pl.BlockSpec((64, 128), lambda i: (base_blk + i, 0)),
        out_shape=jax.ShapeDtypeStruct((vocab, 128), jnp.float32),
        input_output_aliases={0: 0},
    )


def _lane_gather(v, idx):
    dn = lax.GatherDimensionNumbers(
        offset_dims=(), collapsed_slice_dims=(0,), start_index_map=(0,)
    )
    return lax.gather(
        v,
        idx[:, None],
        dn,
        slice_sizes=(1,),
        mode=lax.GatherScatterMode.PROMISE_IN_BOUNDS,
    )


def _make_sc(batch, seq_len, c, vocab):
    info = plsc.get_sparse_core_info()
    nc, ns = info.num_cores, info.num_subcores
    nw = nc * ns
    rpw = batch // nw
    cr = 16
    n_chunks = rpw // cr
    tpc = cr * seq_len
    steps = seq_len // 2

    mesh = plsc.VectorSubcoreMesh(core_axis_name="c", subcore_axis_name="s")

    @functools.partial(
        pl.kernel,
        mesh=mesh,
        out_type=jax.ShapeDtypeStruct((batch * c,), jnp.float32),
        scratch_types=[
            pltpu.VMEM((seq_len, cr), jnp.int32),
            pltpu.VMEM((seq_len, cr), jnp.int32),
            pltpu.VMEM((tpc,), jnp.int32),
            pltpu.VMEM((tpc,), jnp.int32),
            pltpu.VMEM((tpc, _DPAD), jnp.float32),
            pltpu.VMEM((tpc, _DPAD), jnp.float32),
            pltpu.VMEM((rpw * c,), jnp.float32),
            pltpu.SemaphoreType.DMA,
            pltpu.SemaphoreType.DMA,
            pltpu.SemaphoreType.DMA,
            pltpu.SemaphoreType.DMA,
        ],
        compiler_params=pltpu.CompilerParams(
            needs_layout_passes=False, use_tc_tiling_on_sc=False
        ),
    )
    def sc_kernel(
        xt_hbm, tw_hbm, out_hbm,
        ibt0, ibt1, ib0, ib1, db0, db1, outbuf,
        sem_t0, sem_t1, sem_g0, sem_g1,
    ):
        wid = lax.axis_index("s") * nc + lax.axis_index("c")
        row0 = wid * rpw
        iota = lax.iota(jnp.int32, 16)
        riota = lax.shift_right_logical(iota, 3)
        cpat = lax.bitwise_and(iota, 7)
        opat = lax.bitwise_and(iota, 3)
        fold8 = lax.bitwise_and(iota + 8, 15)
        omask = iota < 4
        riota200 = iota * seq_len

        def issue_ibt(g, ibt, sem):
            col = row0 + g * cr
            pltpu.async_copy(xt_hbm.at[:, pl.ds(col, cr)], ibt, sem)

        def wait_ibt(ibt, sem):
            pltpu.make_async_copy(
                xt_hbm.at[:, pl.ds(0, cr)], ibt, sem
            ).wait()

        def transpose(ibt, ib):
            def body(l, _):
                v = lax.shift_left(ibt[l], 4)
                plsc.store_scatter(ib, [riota200 + l], v)
                return 0

            lax.fori_loop(0, seq_len, body, 0)

        def issue_gathers(ib, db, sem):
            def body(r, _):
                pltpu.async_copy(
                    tw_hbm.at[ib.at[pl.ds(r * seq_len, seq_len)]],
                    db.at[pl.ds(r * seq_len, seq_len), :],
                    sem,
                )
                return 0

            lax.fori_loop(0, cr, body, 0)

        def wait_gathers(ib, db, sem):
            def body(r, _):
                pltpu.make_async_copy(
                    tw_hbm.at[ib.at[pl.ds(r * seq_len, seq_len)]],
                    db.at[pl.ds(r * seq_len, seq_len), :],
                    sem,
                ).wait()
                return 0

            lax.fori_loop(0, cr, body, 0)

        def accum(g, db):
            def row_body(r, _):
                base = r * seq_len
                acc = jnp.zeros((16,), jnp.float32)
                for s in range(steps):
                    ridx = riota + (base + 2 * s)
                    acc = acc + plsc.load_gather(db, [ridx, cpat])
                a1 = acc + _lane_gather(acc, fold8)
                oidx = (g * cr + r) * c + opat
                plsc.store_scatter(outbuf, [oidx], a1, mask=omask)
                return 0

            lax.fori_loop(0, cr, row_body, 0)

        issue_ibt(0, ibt0, sem_t0)
        wait_ibt(ibt0, sem_t0)
        transpose(ibt0, ib0)
        issue_gathers(ib0, db0, sem_g0)
        issue_ibt(1, ibt1, sem_t1)

        def body2(k, _):
            g0 = 2 * k
            wait_gathers(ib0, db0, sem_g0)
            wait_ibt(ibt1, sem_t1)
            transpose(ibt1, ib1)
            issue_gathers(ib1, db1, sem_g1)

            @pl.when(g0 + 2 < n_chunks)
            def _():
                issue_ibt(g0 + 2, ibt0, sem_t0)

            accum(g0, db0)

            wait_gathers(ib1, db1, sem_g1)

            @pl.when(g0 + 2 < n_chunks)
            def _():
                wait_ibt(ibt0, sem_t0)
                transpose(ibt0, ib0)
                issue_gathers(ib0, db0, sem_g0)

            @pl.when(g0 + 3 < n_chunks)
            def _():
                issue_ibt(g0 + 3, ibt1, sem_t1)

            accum(g0 + 1, db1)
            return 0

        lax.fori_loop(0, n_chunks // 2, body2, 0)
        pltpu.sync_copy(outbuf, out_hbm.at[pl.ds(row0 * c, rpw * c)])

    return sc_kernel


def kernel(x, table, W, b):
    batch, seq_len = x.shape
    vocab, d = table.shape
    c = W.shape[1]
    b1 = b.reshape(1, c)
    tail = vocab - (vocab // _ROWS) * _ROWS
    tw128 = _make_tw_main(vocab, d, c, seq_len)(table.T, W, b1)
    if tail:
        ttail = lax.slice(table, (vocab - tail, 0), (vocab, d))
        tw128 = _make_tw_tail(vocab, d, c, seq_len, tail)(
            tw128, ttail, W, b1
        )
    tw = tw128.reshape(vocab * (128 // _DPAD), _DPAD)
    out_flat = _make_sc(batch, seq_len, c, vocab)(x.T, tw)
    return out_flat.reshape(batch, c)
